# Initial kernel scaffold; baseline (speedup 1.0000x reference)
import jax
import jax.numpy as jnp
from jax.experimental import pallas as pl


def kernel(x, gate_W, gate_b, W1s, b1s, W2s, b2s, gamma, beta):
    raise NotImplementedError("write your pallas kernel here")


# speedup vs baseline: 1.6330x; 1.6330x over previous
import functools

import jax
import jax.numpy as jnp
from jax import lax
from jax.experimental import pallas as pl
from jax.experimental.pallas import tpu as pltpu
from jax.experimental.pallas import tpu_sc as plsc

S = 2048
H = 768
E = 8
F = 3072
TM = 128
NPAD = S * 2 + E * TM
NTILES = NPAD // TM
LANES = 128

_NC = 2
_NS = 16
_NW = _NC * _NS
_TPW = S // _NW


HP = H // 2


def _pack_bf16(v):
    u = lax.bitcast_convert_type(v, jnp.uint32)
    r = (u + jnp.uint32(0x8000)) >> 16
    packed = (r[..., :HP] << 16) | r[..., HP:]
    return lax.bitcast_convert_type(packed, jnp.float32)


def _unpack_bf16(p):
    u = lax.bitcast_convert_type(p, jnp.uint32)
    a = lax.bitcast_convert_type(u & jnp.uint32(0xFFFF0000), jnp.float32)
    b = lax.bitcast_convert_type(u << 16, jnp.float32)
    return jnp.concatenate([a, b], axis=-1)


def _gating_body(x_ref, gw_ref, gb_ref, pos0_ref, pos1_ref, w0_ref, w1_ref,
                 te_ref, xp_ref):
    x = x_ref[...]
    xp_ref[...] = _pack_bf16(x)
    gw = gw_ref[...]
    logits = jnp.dot(x, gw, preferred_element_type=jnp.float32)
    logits = logits + gb_ref[...]
    lane = lax.broadcasted_iota(jnp.int32, (S, LANES), 1)
    neg = jnp.float32(-3.0e38)
    logits = jnp.where(lane < E, logits, neg)

    m0 = jnp.max(logits, axis=1, keepdims=True)
    idx0 = jnp.min(jnp.where(logits == m0, lane, LANES), axis=1,
                   keepdims=True)
    logits2 = jnp.where(lane == idx0, neg, logits)
    m1 = jnp.max(logits2, axis=1, keepdims=True)
    idx1 = jnp.min(jnp.where(logits2 == m1, lane, LANES), axis=1,
                   keepdims=True)

    e1 = jnp.exp(m1 - m0)
    w1 = e1 / (1.0 + e1)
    w0_ref[...] = 1.0 - w1
    w1_ref[...] = w1

    oh0 = (lane == idx0).astype(jnp.int32)
    oh1 = (lane == idx1).astype(jnp.int32)

    c0 = oh0
    c1 = oh1
    shift = 1
    while shift < S:
        zpad0 = jnp.zeros((shift, LANES), jnp.int32)
        c0 = c0 + jnp.concatenate([zpad0, c0[: S - shift]], axis=0)
        c1 = c1 + jnp.concatenate([zpad0, c1[: S - shift]], axis=0)
        shift *= 2

    counts = (c0[S - 1:] + c1[S - 1:])
    padded = ((counts + (TM - 1)) // TM) * TM
    pc = padded
    sh = 1
    while sh < E:
        zpad = jnp.zeros((1, sh), jnp.int32)
        pc = pc + jnp.concatenate([zpad, pc[:, : LANES - sh]], axis=1)
        sh *= 2
    pc_excl = pc - padded

    rank0 = (c0 - oh0) + (c1 - oh1)
    rank1 = c0 + (c1 - oh1)
    pos0 = jnp.sum(oh0 * (pc_excl + rank0), axis=1, keepdims=True)
    pos1 = jnp.sum(oh1 * (pc_excl + rank1), axis=1, keepdims=True)
    pos0_ref[...] = pos0
    pos1_ref[...] = pos1

    jrow = lax.broadcasted_iota(jnp.int32, (LANES, LANES), 0) * TM
    pcb = jnp.broadcast_to(pc, (LANES, LANES))
    lane2 = lax.broadcasted_iota(jnp.int32, (LANES, LANES), 1)
    te = jnp.sum(((pcb <= jrow) & (lane2 < E)).astype(jnp.int32), axis=1,
                 keepdims=True)
    te_ref[...] = jnp.minimum(te, E - 1)


def _gating(x2d, gate_W, gate_b):
    gw = jnp.zeros((H, LANES), jnp.float32).at[:, :E].set(gate_W)
    gb = jnp.zeros((1, LANES), jnp.float32).at[0, :E].set(gate_b)
    outs = pl.pallas_call(
        _gating_body,
        out_shape=(
            jax.ShapeDtypeStruct((S, 1), jnp.int32),
            jax.ShapeDtypeStruct((S, 1), jnp.int32),
            jax.ShapeDtypeStruct((S, 1), jnp.float32),
            jax.ShapeDtypeStruct((S, 1), jnp.float32),
            jax.ShapeDtypeStruct((LANES, 1), jnp.int32),
            jax.ShapeDtypeStruct((S, HP), jnp.float32),
        ),
    )(x2d, gw, gb)
    return outs


def _sc_mesh():
    return plsc.VectorSubcoreMesh(core_axis_name="c", subcore_axis_name="s")


def _scatter_body(x_hbm, pos0_hbm, pos1_hbm, xs_hbm, idx0_v, idx1_v, rows_v,
                  sem):
    wid = lax.axis_index("s") * _NC + lax.axis_index("c")
    tb = wid * _TPW
    pltpu.sync_copy(x_hbm.at[pl.ds(tb, _TPW)], rows_v)
    pltpu.sync_copy(pos0_hbm.at[pl.ds(tb, _TPW)], idx0_v)
    pltpu.sync_copy(pos1_hbm.at[pl.ds(tb, _TPW)], idx1_v)
    pltpu.async_copy(rows_v, xs_hbm.at[idx0_v], sem).wait()
    pltpu.async_copy(rows_v, xs_hbm.at[idx1_v], sem).wait()


def _sc_scatter_x(xp, pos0, pos1):
    k = pl.kernel(
        _scatter_body,
        out_type=jax.ShapeDtypeStruct((NPAD, HP), jnp.float32),
        mesh=_sc_mesh(),
        scratch_types=[
            pltpu.VMEM((_TPW,), jnp.int32),
            pltpu.VMEM((_TPW,), jnp.int32),
            pltpu.VMEM((_TPW, HP), jnp.float32),
            pltpu.SemaphoreType.DMA,
        ],
    )
    return k(xp, pos0, pos1)


def _gather_body(ys_hbm, pos0_hbm, pos1_hbm, y0_hbm, y1_hbm, idx0_v, idx1_v,
                 rows0_v, rows1_v, sem):
    wid = lax.axis_index("s") * _NC + lax.axis_index("c")
    tb = wid * _TPW
    pltpu.sync_copy(pos0_hbm.at[pl.ds(tb, _TPW)], idx0_v)
    pltpu.sync_copy(pos1_hbm.at[pl.ds(tb, _TPW)], idx1_v)
    pltpu.async_copy(ys_hbm.at[idx0_v], rows0_v, sem).wait()
    pltpu.async_copy(ys_hbm.at[idx1_v], rows1_v, sem).wait()
    pltpu.sync_copy(rows0_v, y0_hbm.at[pl.ds(tb, _TPW)])
    pltpu.sync_copy(rows1_v, y1_hbm.at[pl.ds(tb, _TPW)])


def _sc_gather_y(y_sorted, pos0, pos1):
    k = pl.kernel(
        _gather_body,
        out_type=(
            jax.ShapeDtypeStruct((S, HP), jnp.float32),
            jax.ShapeDtypeStruct((S, HP), jnp.float32),
        ),
        mesh=_sc_mesh(),
        scratch_types=[
            pltpu.VMEM((_TPW,), jnp.int32),
            pltpu.VMEM((_TPW,), jnp.int32),
            pltpu.VMEM((_TPW, HP), jnp.float32),
            pltpu.VMEM((_TPW, HP), jnp.float32),
            pltpu.SemaphoreType.DMA,
        ],
    )
    return k(y_sorted, pos0, pos1)


def _mm_body(te_ref, xs_ref, w1_ref, w2_ref, b1_ref, b2_ref, y_ref, acc_ref):
    xb = _unpack_bf16(xs_ref[...]).astype(jnp.bfloat16)
    acc_ref[...] = jnp.broadcast_to(b2_ref[0, 0], (TM, H))
    for fc in range(F // H):
        sl = pl.ds(fc * H, H)
        w1c = w1_ref[0, :, sl].astype(jnp.bfloat16)
        h = jnp.dot(xb, w1c, preferred_element_type=jnp.float32)
        h = jnp.maximum(h + b1_ref[0, 0, sl], 0.0).astype(jnp.bfloat16)
        w2c = w2_ref[0, sl, :].astype(jnp.bfloat16)
        acc_ref[...] += jnp.dot(h, w2c, preferred_element_type=jnp.float32)
    y_ref[...] = _pack_bf16(acc_ref[...])


def _grouped_mm(x_sorted, te, W1b, W2b, b1s, b2s):
    grid_spec = pltpu.PrefetchScalarGridSpec(
        num_scalar_prefetch=1,
        grid=(NTILES,),
        in_specs=[
            pl.BlockSpec((TM, HP), lambda i, te: (i, 0)),
            pl.BlockSpec((1, H, F), lambda i, te: (te[i], 0, 0)),
            pl.BlockSpec((1, F, H), lambda i, te: (te[i], 0, 0)),
            pl.BlockSpec((1, 1, F), lambda i, te: (te[i], 0, 0)),
            pl.BlockSpec((1, 1, H), lambda i, te: (te[i], 0, 0)),
        ],
        out_specs=pl.BlockSpec((TM, HP), lambda i, te: (i, 0)),
        scratch_shapes=[pltpu.VMEM((TM, H), jnp.float32)],
    )
    return pl.pallas_call(
        _mm_body,
        grid_spec=grid_spec,
        out_shape=jax.ShapeDtypeStruct((NPAD, HP), jnp.float32),
    )(te, x_sorted, W1b, W2b, b1s.reshape(E, 1, F), b2s.reshape(E, 1, H))


def _combine_body(x_ref, y0_ref, y1_ref, w0_ref, w1_ref, g_ref, b_ref,
                  out_ref):
    y0 = _unpack_bf16(y0_ref[...])
    y1 = _unpack_bf16(y1_ref[...])
    y = x_ref[...] + w0_ref[...] * y0 + w1_ref[...] * y1
    mu = jnp.mean(y, axis=1, keepdims=True)
    d = y - mu
    var = jnp.mean(d * d, axis=1, keepdims=True)
    out_ref[...] = d * lax.rsqrt(var + 1e-5) * g_ref[...] + b_ref[...]


def _combine(x2d, y0, y1, w0, w1, gamma, beta):
    bt = 256
    grid_spec = pl.GridSpec(
        grid=(S // bt,),
        in_specs=[
            pl.BlockSpec((bt, H), lambda i: (i, 0)),
            pl.BlockSpec((bt, HP), lambda i: (i, 0)),
            pl.BlockSpec((bt, HP), lambda i: (i, 0)),
            pl.BlockSpec((bt, 1), lambda i: (i, 0)),
            pl.BlockSpec((bt, 1), lambda i: (i, 0)),
            pl.BlockSpec((1, H), lambda i: (0, 0)),
            pl.BlockSpec((1, H), lambda i: (0, 0)),
        ],
        out_specs=pl.BlockSpec((bt, H), lambda i: (i, 0)),
    )
    return pl.pallas_call(
        _combine_body,
        grid_spec=grid_spec,
        out_shape=jax.ShapeDtypeStruct((S, H), jnp.float32),
    )(x2d, y0, y1, w0, w1, gamma.reshape(1, H), beta.reshape(1, H))


def kernel(x, gate_W, gate_b, W1s, b1s, W2s, b2s, gamma, beta):
    x2d = x.reshape(S, H)
    pos0, pos1, w0, w1, te_col, xp = _gating(x2d, gate_W, gate_b)
    te = te_col.reshape(LANES)[:NTILES]
    p0 = pos0.reshape(S)
    p1 = pos1.reshape(S)

    x_sorted = _sc_scatter_x(xp, p0, p1)
    y_sorted = _grouped_mm(x_sorted, te, W1s, W2s, b1s, b2s)
    y0, y1 = _sc_gather_y(y_sorted, p0, p1)
    out = _combine(x2d, y0, y1, w0, w1, gamma, beta)
    return out.reshape(x.shape)


# speedup vs baseline: 1.7993x; 1.1018x over previous
import functools

import jax
import jax.numpy as jnp
from jax import lax
from jax.experimental import pallas as pl
from jax.experimental.pallas import tpu as pltpu
from jax.experimental.pallas import tpu_sc as plsc

S = 2048
H = 768
E = 8
F = 3072
TM = 256
NPAD = S * 2 + E * TM
NTILES = NPAD // TM
LANES = 128

_NC = 2
_NS = 16
_NW = _NC * _NS
_TPW = S // _NW


HP = H // 2


def _pack_bf16(v):
    u = lax.bitcast_convert_type(v, jnp.uint32)
    r = (u + jnp.uint32(0x8000)) >> 16
    packed = (r[..., :HP] << 16) | r[..., HP:]
    return lax.bitcast_convert_type(packed, jnp.float32)


def _unpack_bf16(p):
    u = lax.bitcast_convert_type(p, jnp.uint32)
    a = lax.bitcast_convert_type(u & jnp.uint32(0xFFFF0000), jnp.float32)
    b = lax.bitcast_convert_type(u << 16, jnp.float32)
    return jnp.concatenate([a, b], axis=-1)


def _gating_body(x_ref, gw_ref, gb_ref, pos0_ref, pos1_ref, w0_ref, w1_ref,
                 te_ref, xp_ref):
    x = x_ref[...]
    xp_ref[...] = _pack_bf16(x)
    gw = gw_ref[...]
    logits = jnp.dot(x, gw, preferred_element_type=jnp.float32)
    logits = logits + gb_ref[...]
    lane = lax.broadcasted_iota(jnp.int32, (S, LANES), 1)
    neg = jnp.float32(-3.0e38)
    logits = jnp.where(lane < E, logits, neg)

    m0 = jnp.max(logits, axis=1, keepdims=True)
    idx0 = jnp.min(jnp.where(logits == m0, lane, LANES), axis=1,
                   keepdims=True)
    logits2 = jnp.where(lane == idx0, neg, logits)
    m1 = jnp.max(logits2, axis=1, keepdims=True)
    idx1 = jnp.min(jnp.where(logits2 == m1, lane, LANES), axis=1,
                   keepdims=True)

    e1 = jnp.exp(m1 - m0)
    w1 = e1 / (1.0 + e1)
    w0_ref[...] = 1.0 - w1
    w1_ref[...] = w1

    oh0 = (lane == idx0).astype(jnp.int32)
    oh1 = (lane == idx1).astype(jnp.int32)

    c0 = oh0
    c1 = oh1
    shift = 1
    while shift < S:
        zpad0 = jnp.zeros((shift, LANES), jnp.int32)
        c0 = c0 + jnp.concatenate([zpad0, c0[: S - shift]], axis=0)
        c1 = c1 + jnp.concatenate([zpad0, c1[: S - shift]], axis=0)
        shift *= 2

    counts = (c0[S - 1:] + c1[S - 1:])
    padded = ((counts + (TM - 1)) // TM) * TM
    pc = padded
    sh = 1
    while sh < E:
        zpad = jnp.zeros((1, sh), jnp.int32)
        pc = pc + jnp.concatenate([zpad, pc[:, : LANES - sh]], axis=1)
        sh *= 2
    pc_excl = pc - padded

    rank0 = (c0 - oh0) + (c1 - oh1)
    rank1 = c0 + (c1 - oh1)
    pos0 = jnp.sum(oh0 * (pc_excl + rank0), axis=1, keepdims=True)
    pos1 = jnp.sum(oh1 * (pc_excl + rank1), axis=1, keepdims=True)
    pos0_ref[...] = pos0
    pos1_ref[...] = pos1

    jrow = lax.broadcasted_iota(jnp.int32, (LANES, LANES), 0) * TM
    pcb = jnp.broadcast_to(pc, (LANES, LANES))
    lane2 = lax.broadcasted_iota(jnp.int32, (LANES, LANES), 1)
    te = jnp.sum(((pcb <= jrow) & (lane2 < E)).astype(jnp.int32), axis=1,
                 keepdims=True)
    te_ref[...] = jnp.minimum(te, E - 1)


def _gating(x2d, gate_W, gate_b):
    gw = jnp.zeros((H, LANES), jnp.float32).at[:, :E].set(gate_W)
    gb = jnp.zeros((1, LANES), jnp.float32).at[0, :E].set(gate_b)
    outs = pl.pallas_call(
        _gating_body,
        out_shape=(
            jax.ShapeDtypeStruct((S, 1), jnp.int32),
            jax.ShapeDtypeStruct((S, 1), jnp.int32),
            jax.ShapeDtypeStruct((S, 1), jnp.float32),
            jax.ShapeDtypeStruct((S, 1), jnp.float32),
            jax.ShapeDtypeStruct((LANES, 1), jnp.int32),
            jax.ShapeDtypeStruct((S, HP), jnp.float32),
        ),
    )(x2d, gw, gb)
    return outs


def _sc_mesh():
    return plsc.VectorSubcoreMesh(core_axis_name="c", subcore_axis_name="s")


def _scatter_body(x_hbm, pos0_hbm, pos1_hbm, xs_hbm, idx0_v, idx1_v, rows_v,
                  sem):
    wid = lax.axis_index("s") * _NC + lax.axis_index("c")
    tb = wid * _TPW
    pltpu.sync_copy(x_hbm.at[pl.ds(tb, _TPW)], rows_v)
    pltpu.sync_copy(pos0_hbm.at[pl.ds(tb, _TPW)], idx0_v)
    pltpu.sync_copy(pos1_hbm.at[pl.ds(tb, _TPW)], idx1_v)
    pltpu.async_copy(rows_v, xs_hbm.at[idx0_v], sem).wait()
    pltpu.async_copy(rows_v, xs_hbm.at[idx1_v], sem).wait()


def _sc_scatter_x(xp, pos0, pos1):
    k = pl.kernel(
        _scatter_body,
        out_type=jax.ShapeDtypeStruct((NPAD, HP), jnp.float32),
        mesh=_sc_mesh(),
        scratch_types=[
            pltpu.VMEM((_TPW,), jnp.int32),
            pltpu.VMEM((_TPW,), jnp.int32),
            pltpu.VMEM((_TPW, HP), jnp.float32),
            pltpu.SemaphoreType.DMA,
        ],
    )
    return k(xp, pos0, pos1)


def _gather_body(ys_hbm, pos0_hbm, pos1_hbm, y0_hbm, y1_hbm, idx0_v, idx1_v,
                 rows0_v, rows1_v, sem):
    wid = lax.axis_index("s") * _NC + lax.axis_index("c")
    tb = wid * _TPW
    pltpu.sync_copy(pos0_hbm.at[pl.ds(tb, _TPW)], idx0_v)
    pltpu.sync_copy(pos1_hbm.at[pl.ds(tb, _TPW)], idx1_v)
    pltpu.async_copy(ys_hbm.at[idx0_v], rows0_v, sem).wait()
    pltpu.async_copy(ys_hbm.at[idx1_v], rows1_v, sem).wait()
    pltpu.sync_copy(rows0_v, y0_hbm.at[pl.ds(tb, _TPW)])
    pltpu.sync_copy(rows1_v, y1_hbm.at[pl.ds(tb, _TPW)])


def _sc_gather_y(y_sorted, pos0, pos1):
    k = pl.kernel(
        _gather_body,
        out_type=(
            jax.ShapeDtypeStruct((S, HP), jnp.float32),
            jax.ShapeDtypeStruct((S, HP), jnp.float32),
        ),
        mesh=_sc_mesh(),
        scratch_types=[
            pltpu.VMEM((_TPW,), jnp.int32),
            pltpu.VMEM((_TPW,), jnp.int32),
            pltpu.VMEM((_TPW, HP), jnp.float32),
            pltpu.VMEM((_TPW, HP), jnp.float32),
            pltpu.SemaphoreType.DMA,
        ],
    )
    return k(y_sorted, pos0, pos1)


def _mm_body(te_ref, xs_ref, w1_ref, w2_ref, b1_ref, b2_ref, y_ref, acc_ref):
    xb = _unpack_bf16(xs_ref[...]).astype(jnp.bfloat16)
    acc_ref[...] = jnp.broadcast_to(b2_ref[0, 0], (TM, H))
    for fc in range(F // H):
        sl = pl.ds(fc * H, H)
        w1c = w1_ref[0, :, sl].astype(jnp.bfloat16)
        h = jnp.dot(xb, w1c, preferred_element_type=jnp.float32)
        h = jnp.maximum(h + b1_ref[0, 0, sl], 0.0).astype(jnp.bfloat16)
        w2c = w2_ref[0, sl, :].astype(jnp.bfloat16)
        acc_ref[...] += jnp.dot(h, w2c, preferred_element_type=jnp.float32)
    y_ref[...] = _pack_bf16(acc_ref[...])


def _grouped_mm(x_sorted, te, W1b, W2b, b1s, b2s):
    grid_spec = pltpu.PrefetchScalarGridSpec(
        num_scalar_prefetch=1,
        grid=(NTILES,),
        in_specs=[
            pl.BlockSpec((TM, HP), lambda i, te: (i, 0)),
            pl.BlockSpec((1, H, F), lambda i, te: (te[i], 0, 0)),
            pl.BlockSpec((1, F, H), lambda i, te: (te[i], 0, 0)),
            pl.BlockSpec((1, 1, F), lambda i, te: (te[i], 0, 0)),
            pl.BlockSpec((1, 1, H), lambda i, te: (te[i], 0, 0)),
        ],
        out_specs=pl.BlockSpec((TM, HP), lambda i, te: (i, 0)),
        scratch_shapes=[pltpu.VMEM((TM, H), jnp.float32)],
    )
    return pl.pallas_call(
        _mm_body,
        grid_spec=grid_spec,
        out_shape=jax.ShapeDtypeStruct((NPAD, HP), jnp.float32),
    )(te, x_sorted, W1b, W2b, b1s.reshape(E, 1, F), b2s.reshape(E, 1, H))


def _combine_body(x_ref, y0_ref, y1_ref, w0_ref, w1_ref, g_ref, b_ref,
                  out_ref):
    y0 = _unpack_bf16(y0_ref[...])
    y1 = _unpack_bf16(y1_ref[...])
    y = x_ref[...] + w0_ref[...] * y0 + w1_ref[...] * y1
    mu = jnp.mean(y, axis=1, keepdims=True)
    d = y - mu
    var = jnp.mean(d * d, axis=1, keepdims=True)
    out_ref[...] = d * lax.rsqrt(var + 1e-5) * g_ref[...] + b_ref[...]


def _combine(x2d, y0, y1, w0, w1, gamma, beta):
    bt = 256
    grid_spec = pl.GridSpec(
        grid=(S // bt,),
        in_specs=[
            pl.BlockSpec((bt, H), lambda i: (i, 0)),
            pl.BlockSpec((bt, HP), lambda i: (i, 0)),
            pl.BlockSpec((bt, HP), lambda i: (i, 0)),
            pl.BlockSpec((bt, 1), lambda i: (i, 0)),
            pl.BlockSpec((bt, 1), lambda i: (i, 0)),
            pl.BlockSpec((1, H), lambda i: (0, 0)),
            pl.BlockSpec((1, H), lambda i: (0, 0)),
        ],
        out_specs=pl.BlockSpec((bt, H), lambda i: (i, 0)),
    )
    return pl.pallas_call(
        _combine_body,
        grid_spec=grid_spec,
        out_shape=jax.ShapeDtypeStruct((S, H), jnp.float32),
    )(x2d, y0, y1, w0, w1, gamma.reshape(1, H), beta.reshape(1, H))


def kernel(x, gate_W, gate_b, W1s, b1s, W2s, b2s, gamma, beta):
    x2d = x.reshape(S, H)
    pos0, pos1, w0, w1, te_col, xp = _gating(x2d, gate_W, gate_b)
    te = te_col.reshape(LANES)[:NTILES]
    p0 = pos0.reshape(S)
    p1 = pos1.reshape(S)

    x_sorted = _sc_scatter_x(xp, p0, p1)
    y_sorted = _grouped_mm(x_sorted, te, W1s, W2s, b1s, b2s)
    y0, y1 = _sc_gather_y(y_sorted, p0, p1)
    out = _combine(x2d, y0, y1, w0, w1, gamma, beta)
    return out.reshape(x.shape)


# speedup vs baseline: 1.8034x; 1.0023x over previous
import jax
import jax.numpy as jnp
from jax import lax
from jax.experimental import pallas as pl
from jax.experimental.pallas import tpu as pltpu
from jax.experimental.pallas import tpu_sc as plsc

S = 2048
H = 768
E = 8
F = 3072
TM = 256
NPAD = S * 2 + E * TM
NTILES = NPAD // TM
LANES = 128

_NC = 2
_NS = 16
_NW = _NC * _NS
_TPW = S // _NW


HP = H // 2


def _pack_bf16(v):
    u = lax.bitcast_convert_type(v, jnp.uint32)
    r = (u + jnp.uint32(0x8000)) >> 16
    packed = (r[..., :HP] << 16) | r[..., HP:]
    return lax.bitcast_convert_type(packed, jnp.float32)


def _unpack_bf16(p):
    u = lax.bitcast_convert_type(p, jnp.uint32)
    a = lax.bitcast_convert_type(u & jnp.uint32(0xFFFF0000), jnp.float32)
    b = lax.bitcast_convert_type(u << 16, jnp.float32)
    return jnp.concatenate([a, b], axis=-1)


def _gating_body(x_ref, gw_ref, gb_ref, pos0_ref, pos1_ref, w0_ref, w1_ref,
                 te_ref, xp_ref):
    x = x_ref[...]
    xp_ref[...] = _pack_bf16(x)
    gw = gw_ref[...]
    logits = jnp.dot(x, gw, preferred_element_type=jnp.float32)
    logits = logits + gb_ref[...]
    lane = lax.broadcasted_iota(jnp.int32, (S, LANES), 1)
    neg = jnp.float32(-3.0e38)
    logits = jnp.where(lane < E, logits, neg)

    m0 = jnp.max(logits, axis=1, keepdims=True)
    idx0 = jnp.min(jnp.where(logits == m0, lane, LANES), axis=1,
                   keepdims=True)
    logits2 = jnp.where(lane == idx0, neg, logits)
    m1 = jnp.max(logits2, axis=1, keepdims=True)
    idx1 = jnp.min(jnp.where(logits2 == m1, lane, LANES), axis=1,
                   keepdims=True)

    e1 = jnp.exp(m1 - m0)
    w1 = e1 / (1.0 + e1)
    w0_ref[...] = 1.0 - w1
    w1_ref[...] = w1

    oh0 = (lane == idx0).astype(jnp.int32)
    oh1 = (lane == idx1).astype(jnp.int32)

    c0 = oh0
    c1 = oh1
    shift = 1
    while shift < S:
        zpad0 = jnp.zeros((shift, LANES), jnp.int32)
        c0 = c0 + jnp.concatenate([zpad0, c0[: S - shift]], axis=0)
        c1 = c1 + jnp.concatenate([zpad0, c1[: S - shift]], axis=0)
        shift *= 2

    counts = (c0[S - 1:] + c1[S - 1:])
    padded = ((counts + (TM - 1)) // TM) * TM
    pc = padded
    sh = 1
    while sh < E:
        zpad = jnp.zeros((1, sh), jnp.int32)
        pc = pc + jnp.concatenate([zpad, pc[:, : LANES - sh]], axis=1)
        sh *= 2
    pc_excl = pc - padded

    rank0 = (c0 - oh0) + (c1 - oh1)
    rank1 = c0 + (c1 - oh1)
    pos0 = jnp.sum(oh0 * (pc_excl + rank0), axis=1, keepdims=True)
    pos1 = jnp.sum(oh1 * (pc_excl + rank1), axis=1, keepdims=True)
    pos0_ref[...] = pos0
    pos1_ref[...] = pos1

    jrow = lax.broadcasted_iota(jnp.int32, (LANES, LANES), 0) * TM
    pcb = jnp.broadcast_to(pc, (LANES, LANES))
    lane2 = lax.broadcasted_iota(jnp.int32, (LANES, LANES), 1)
    te = jnp.sum(((pcb <= jrow) & (lane2 < E)).astype(jnp.int32), axis=1,
                 keepdims=True)
    te_ref[...] = jnp.minimum(te, E - 1)


def _gating(x2d, gate_W, gate_b):
    gw = jnp.zeros((H, LANES), jnp.float32).at[:, :E].set(gate_W)
    gb = jnp.zeros((1, LANES), jnp.float32).at[0, :E].set(gate_b)
    outs = pl.pallas_call(
        _gating_body,
        out_shape=(
            jax.ShapeDtypeStruct((S, 1), jnp.int32),
            jax.ShapeDtypeStruct((S, 1), jnp.int32),
            jax.ShapeDtypeStruct((S, 1), jnp.float32),
            jax.ShapeDtypeStruct((S, 1), jnp.float32),
            jax.ShapeDtypeStruct((LANES, 1), jnp.int32),
            jax.ShapeDtypeStruct((S, HP), jnp.float32),
        ),
    )(x2d, gw, gb)
    return outs


def _sc_mesh():
    return plsc.VectorSubcoreMesh(core_axis_name="c", subcore_axis_name="s")


def _scatter_body(x_hbm, pos0_hbm, pos1_hbm, xs_hbm, idx0_v, idx1_v, rows_v,
                  sem):
    wid = lax.axis_index("s") * _NC + lax.axis_index("c")
    tb = wid * _TPW
    pltpu.sync_copy(x_hbm.at[pl.ds(tb, _TPW)], rows_v)
    pltpu.sync_copy(pos0_hbm.at[pl.ds(tb, _TPW)], idx0_v)
    pltpu.sync_copy(pos1_hbm.at[pl.ds(tb, _TPW)], idx1_v)
    pltpu.async_copy(rows_v, xs_hbm.at[idx0_v], sem).wait()
    pltpu.async_copy(rows_v, xs_hbm.at[idx1_v], sem).wait()


def _sc_scatter_x(xp, pos0, pos1):
    k = pl.kernel(
        _scatter_body,
        out_type=jax.ShapeDtypeStruct((NPAD, HP), jnp.float32),
        mesh=_sc_mesh(),
        scratch_types=[
            pltpu.VMEM((_TPW,), jnp.int32),
            pltpu.VMEM((_TPW,), jnp.int32),
            pltpu.VMEM((_TPW, HP), jnp.float32),
            pltpu.SemaphoreType.DMA,
        ],
    )
    return k(xp, pos0, pos1)


def _gather_body(ys_hbm, pos0_hbm, pos1_hbm, y0_hbm, y1_hbm, idx0_v, idx1_v,
                 rows0_v, rows1_v, sem):
    wid = lax.axis_index("s") * _NC + lax.axis_index("c")
    tb = wid * _TPW
    pltpu.sync_copy(pos0_hbm.at[pl.ds(tb, _TPW)], idx0_v)
    pltpu.sync_copy(pos1_hbm.at[pl.ds(tb, _TPW)], idx1_v)
    pltpu.async_copy(ys_hbm.at[idx0_v], rows0_v, sem).wait()
    pltpu.async_copy(ys_hbm.at[idx1_v], rows1_v, sem).wait()
    pltpu.sync_copy(rows0_v, y0_hbm.at[pl.ds(tb, _TPW)])
    pltpu.sync_copy(rows1_v, y1_hbm.at[pl.ds(tb, _TPW)])


def _sc_gather_y(y_sorted, pos0, pos1):
    k = pl.kernel(
        _gather_body,
        out_type=(
            jax.ShapeDtypeStruct((S, HP), jnp.float32),
            jax.ShapeDtypeStruct((S, HP), jnp.float32),
        ),
        mesh=_sc_mesh(),
        scratch_types=[
            pltpu.VMEM((_TPW,), jnp.int32),
            pltpu.VMEM((_TPW,), jnp.int32),
            pltpu.VMEM((_TPW, HP), jnp.float32),
            pltpu.VMEM((_TPW, HP), jnp.float32),
            pltpu.SemaphoreType.DMA,
        ],
    )
    return k(y_sorted, pos0, pos1)


def _mm_body(te_ref, xs_ref, w1_ref, w2_ref, b1_ref, b2_ref, y_ref, acc_ref):
    xb = _unpack_bf16(xs_ref[...]).astype(jnp.bfloat16)
    acc_ref[...] = jnp.broadcast_to(b2_ref[0, 0], (TM, H))
    for fc in range(F // H):
        sl = pl.ds(fc * H, H)
        w1c = w1_ref[0, :, sl].astype(jnp.bfloat16)
        h = jnp.dot(xb, w1c, preferred_element_type=jnp.float32)
        h = jnp.maximum(h + b1_ref[0, 0, sl], 0.0).astype(jnp.bfloat16)
        w2c = w2_ref[0, sl, :].astype(jnp.bfloat16)
        acc_ref[...] += jnp.dot(h, w2c, preferred_element_type=jnp.float32)
    y_ref[...] = _pack_bf16(acc_ref[...])


def _grouped_mm(x_sorted, te, W1b, W2b, b1s, b2s):
    grid_spec = pltpu.PrefetchScalarGridSpec(
        num_scalar_prefetch=1,
        grid=(NTILES,),
        in_specs=[
            pl.BlockSpec((TM, HP), lambda i, te: (i, 0)),
            pl.BlockSpec((1, H, F), lambda i, te: (te[i], 0, 0)),
            pl.BlockSpec((1, F, H), lambda i, te: (te[i], 0, 0)),
            pl.BlockSpec((1, 1, F), lambda i, te: (te[i], 0, 0)),
            pl.BlockSpec((1, 1, H), lambda i, te: (te[i], 0, 0)),
        ],
        out_specs=pl.BlockSpec((TM, HP), lambda i, te: (i, 0)),
        scratch_shapes=[pltpu.VMEM((TM, H), jnp.float32)],
    )
    return pl.pallas_call(
        _mm_body,
        grid_spec=grid_spec,
        out_shape=jax.ShapeDtypeStruct((NPAD, HP), jnp.float32),
    )(te, x_sorted, W1b, W2b, b1s.reshape(E, 1, F), b2s.reshape(E, 1, H))


def _combine_body(x_ref, y0_ref, y1_ref, w0_ref, w1_ref, g_ref, b_ref,
                  out_ref):
    y0 = _unpack_bf16(y0_ref[...])
    y1 = _unpack_bf16(y1_ref[...])
    y = x_ref[...] + w0_ref[...] * y0 + w1_ref[...] * y1
    mu = jnp.mean(y, axis=1, keepdims=True)
    d = y - mu
    var = jnp.mean(d * d, axis=1, keepdims=True)
    out_ref[...] = d * lax.rsqrt(var + 1e-5) * g_ref[...] + b_ref[...]


def _combine(x2d, y0, y1, w0, w1, gamma, beta):
    bt = 256
    grid_spec = pl.GridSpec(
        grid=(S // bt,),
        in_specs=[
            pl.BlockSpec((bt, H), lambda i: (i, 0)),
            pl.BlockSpec((bt, HP), lambda i: (i, 0)),
            pl.BlockSpec((bt, HP), lambda i: (i, 0)),
            pl.BlockSpec((bt, 1), lambda i: (i, 0)),
            pl.BlockSpec((bt, 1), lambda i: (i, 0)),
            pl.BlockSpec((1, H), lambda i: (0, 0)),
            pl.BlockSpec((1, H), lambda i: (0, 0)),
        ],
        out_specs=pl.BlockSpec((bt, H), lambda i: (i, 0)),
    )
    return pl.pallas_call(
        _combine_body,
        grid_spec=grid_spec,
        out_shape=jax.ShapeDtypeStruct((S, H), jnp.float32),
    )(x2d, y0, y1, w0, w1, gamma.reshape(1, H), beta.reshape(1, H))


def kernel(x, gate_W, gate_b, W1s, b1s, W2s, b2s, gamma, beta):
    x2d = x.reshape(S, H)
    pos0, pos1, w0, w1, te_col, xp = _gating(x2d, gate_W, gate_b)
    te = te_col.reshape(LANES)[:NTILES]
    p0 = pos0.reshape(S)
    p1 = pos1.reshape(S)

    x_sorted = _sc_scatter_x(xp, p0, p1)
    y_sorted = _grouped_mm(x_sorted, te, W1s, W2s, b1s, b2s)
    y0, y1 = _sc_gather_y(y_sorted, p0, p1)
    out = _combine(x2d, y0, y1, w0, w1, gamma, beta)
    return out.reshape(x.shape)
